# baseline (device time: 77338 ns/iter reference)
import os

import jax
import jax.numpy as jnp
from jax import lax
from jax.experimental import pallas as pl
from jax.experimental.pallas import tpu as pltpu

N_DEV = 32
B, SQ, DM = 2, 256, 512
DH = 64
H_PER = 4
HQ = 128
ROWS = B * SQ
CHUNK = ROWS // N_DEV
SBLK = 64
NBLK = SQ // SBLK
NTILE = B * NBLK

DO_RS = os.environ.get("KERNEL_NO_RS") != "1"
DO_AG = os.environ.get("KERNEL_NO_AG") != "1"
DO_KV = os.environ.get("KERNEL_NO_KV") != "1"


def kernel(x, Wq, K_ext, V_ext, Wo):
    K_t = K_ext.transpose(0, 1, 3, 2)
    V_t = V_ext.transpose(0, 1, 3, 2)

    def body(x_ref, wq_ref, k_hbm, v_hbm, wo_ref, out_ref,
             k_ref, v_ref, acc_ref, stage_ref, kv_sems,
             rs_send_sems, rs_recv_sems, ag_send_sems, ag_recv_sems):
        me = lax.axis_index("i")

        kv_copies = []
        for t in range(NTILE):
            b, sb = divmod(t, NBLK)
            kv_copies.append(pltpu.make_async_copy(
                k_hbm.at[b, pl.ds(sb * SBLK, SBLK)],
                k_ref.at[b, pl.ds(sb * SBLK, SBLK)],
                kv_sems.at[t]))
            kv_copies.append(pltpu.make_async_copy(
                v_hbm.at[b, pl.ds(sb * SBLK, SBLK)],
                v_ref.at[b, pl.ds(sb * SBLK, SBLK)],
                kv_sems.at[NTILE + t]))
        if DO_KV:
            for cp in kv_copies:
                cp.start()

        barrier_sem = pltpu.get_barrier_semaphore()
        for nbr in (lax.rem(me + 1, N_DEV), lax.rem(me + N_DEV - 1, N_DEV)):
            pl.semaphore_signal(barrier_sem, inc=1, device_id=(nbr,),
                                device_id_type=pl.DeviceIdType.MESH)
        pl.semaphore_wait(barrier_sem, 2)

        stage_ref[pl.ds(me, 1)] = jnp.zeros((1, CHUNK, DM), jnp.float32)

        x2 = x_ref[...].reshape(ROWS, DM)
        q = jnp.dot(x2, wq_ref[...], preferred_element_type=jnp.float32)
        q4 = q.reshape(B, SQ, H_PER, DH)

        sel = (lax.broadcasted_iota(jnp.int32, (HQ, H_PER), 0)
               == me * H_PER
               + lax.broadcasted_iota(jnp.int32, (HQ, H_PER), 1)
               ).astype(jnp.float32)

        for t in range(NTILE):
            b, sb = divmod(t, NBLK)
            if DO_KV:
                kv_copies[2 * t].wait()
                kv_copies[2 * t + 1].wait()
            rows = pl.ds(sb * SBLK, SBLK)
            k4 = lax.dot_general(
                sel, k_ref[b, rows].reshape(SBLK * DH, HQ),
                (((0,), (1,)), ((), ())),
                preferred_element_type=jnp.float32,
            ).reshape(H_PER, SBLK, DH)
            v4 = lax.dot_general(
                sel, v_ref[b, rows].reshape(SBLK * DH, HQ),
                (((0,), (1,)), ((), ())),
                preferred_element_type=jnp.float32,
            ).reshape(H_PER, SBLK, DH)
            ctxs = []
            for h in range(H_PER):
                qh = q4[b, sb * SBLK:(sb + 1) * SBLK, h, :]
                kh = k4[h]
                vh = v4[h]
                s = lax.dot_general(
                    qh, kh, (((1,), (1,)), ((), ())),
                    preferred_element_type=jnp.float32) * 0.125
                w = jnp.exp(s - jnp.max(s, axis=-1, keepdims=True))
                w = w / jnp.sum(w, axis=-1, keepdims=True)
                ctxs.append(jnp.dot(w, vh, preferred_element_type=jnp.float32))
            ctx_blk = jnp.concatenate(ctxs, axis=1)
            pb = jnp.dot(ctx_blk, wo_ref[...],
                         preferred_element_type=jnp.float32)
            c0 = b * (SQ // CHUNK) + sb * (SBLK // CHUNK)
            acc_ref[pl.ds(c0, SBLK // CHUNK)] = pb.reshape(
                SBLK // CHUNK, CHUNK, DM)

            if DO_RS:
                for c in range(c0, c0 + SBLK // CHUNK):

                    @pl.when(c != me)
                    def _send(c=c):
                        pltpu.make_async_remote_copy(
                            src_ref=acc_ref.at[c],
                            dst_ref=stage_ref.at[me],
                            send_sem=rs_send_sems.at[c],
                            recv_sem=rs_recv_sems.at[me],
                            device_id=(c,),
                            device_id_type=pl.DeviceIdType.MESH,
                        ).start()

            if DO_RS and DO_AG:

                @pl.when(t == jnp.minimum(me // (SBLK // CHUNK) + 1,
                                          NTILE - 1))
                def _reduce_and_bcast():
                    for off in range(1, N_DEV):
                        j = lax.rem(me + off, N_DEV)
                        pltpu.make_async_remote_copy(
                            src_ref=acc_ref.at[j],
                            dst_ref=stage_ref.at[j],
                            send_sem=rs_send_sems.at[j],
                            recv_sem=rs_recv_sems.at[j],
                            device_id=(j,),
                            device_id_type=pl.DeviceIdType.MESH,
                        ).wait_recv()
                    red = (acc_ref[pl.ds(me, 1)]
                           + jnp.sum(stage_ref[...], axis=0, keepdims=True))
                    acc_ref[pl.ds(me, 1)] = red
                    for off in range(1, N_DEV):
                        tgt = lax.rem(me + off, N_DEV)
                        pltpu.make_async_remote_copy(
                            src_ref=acc_ref.at[me],
                            dst_ref=acc_ref.at[me],
                            send_sem=ag_send_sems.at[tgt],
                            recv_sem=ag_recv_sems.at[me],
                            device_id=(tgt,),
                            device_id_type=pl.DeviceIdType.MESH,
                        ).start()

        for off in range(1, N_DEV if DO_AG else 1):
            j = lax.rem(me + off, N_DEV)
            pltpu.make_async_remote_copy(
                src_ref=acc_ref.at[j],
                dst_ref=acc_ref.at[j],
                send_sem=ag_send_sems.at[j],
                recv_sem=ag_recv_sems.at[j],
                device_id=(j,),
                device_id_type=pl.DeviceIdType.MESH,
            ).wait_recv()

        if DO_RS:
            for c in range(N_DEV):

                @pl.when(c != me)
                def _drain(c=c):
                    pltpu.make_async_remote_copy(
                        src_ref=acc_ref.at[c],
                        dst_ref=stage_ref.at[me],
                        send_sem=rs_send_sems.at[c],
                        recv_sem=rs_recv_sems.at[me],
                        device_id=(c,),
                        device_id_type=pl.DeviceIdType.MESH,
                    ).wait_send()
        if DO_AG:
            for off in range(1, N_DEV):
                t = lax.rem(me + off, N_DEV)
                pltpu.make_async_remote_copy(
                    src_ref=acc_ref.at[me],
                    dst_ref=acc_ref.at[me],
                    send_sem=ag_send_sems.at[t],
                    recv_sem=ag_recv_sems.at[me],
                    device_id=(t,),
                    device_id_type=pl.DeviceIdType.MESH,
                ).wait_send()

        out_ref[...] = acc_ref[...].reshape(B, SQ, DM)

    return pl.pallas_call(
        body,
        out_shape=jax.ShapeDtypeStruct((B, SQ, DM), jnp.float32),
        in_specs=[
            pl.BlockSpec(memory_space=pltpu.VMEM),
            pl.BlockSpec(memory_space=pltpu.VMEM),
            pl.BlockSpec(memory_space=pl.ANY),
            pl.BlockSpec(memory_space=pl.ANY),
            pl.BlockSpec(memory_space=pltpu.VMEM),
        ],
        out_specs=pl.BlockSpec(memory_space=pltpu.VMEM),
        scratch_shapes=[
            pltpu.VMEM((B, SQ, DH, HQ), jnp.float32),
            pltpu.VMEM((B, SQ, DH, HQ), jnp.float32),
            pltpu.VMEM((N_DEV, CHUNK, DM), jnp.float32),
            pltpu.VMEM((N_DEV, CHUNK, DM), jnp.float32),
            pltpu.SemaphoreType.DMA((2 * NTILE,)),
            pltpu.SemaphoreType.DMA((N_DEV,)),
            pltpu.SemaphoreType.DMA((N_DEV,)),
            pltpu.SemaphoreType.DMA((N_DEV,)),
            pltpu.SemaphoreType.DMA((N_DEV,)),
        ],
        compiler_params=pltpu.CompilerParams(
            collective_id=0,
            vmem_limit_bytes=50 * 1024 * 1024,
        ),
    )(x, Wq, K_t, V_t, Wo)


# device time: 53522 ns/iter; 1.4450x vs baseline; 1.4450x over previous
import os

import jax
import jax.numpy as jnp
from jax import lax
from jax.experimental import pallas as pl
from jax.experimental.pallas import tpu as pltpu

N_DEV = 32
B, SQ, DM = 2, 256, 512
DH = 64
H_PER = 4
HQ = 128
ROWS = B * SQ
CHUNK = ROWS // N_DEV
SBLK = 64
NBLK = SQ // SBLK
NTILE = B * NBLK

DO_RS = os.environ.get("KERNEL_NO_RS") != "1"
DO_AG = os.environ.get("KERNEL_NO_AG") != "1"
DO_KV = os.environ.get("KERNEL_NO_KV") != "1"


def kernel(x, Wq, K_ext, V_ext, Wo):
    K_t = K_ext.transpose(0, 1, 3, 2)
    V_t = V_ext.transpose(0, 1, 3, 2)

    def body(x_ref, wq_ref, k_hbm, v_hbm, wo_ref, out_ref,
             k_ref, v_ref, acc_ref, stage_ref, kv_sems,
             rs_send_sems, rs_recv_sems, ag_send_sems, ag_recv_sems):
        me = lax.axis_index("i")

        kv_copies = []
        for t in range(NTILE):
            b, sb = divmod(t, NBLK)
            kv_copies.append(pltpu.make_async_copy(
                k_hbm.at[b, pl.ds(sb * SBLK, SBLK)],
                k_ref.at[b, pl.ds(sb * SBLK, SBLK)],
                kv_sems.at[t]))
            kv_copies.append(pltpu.make_async_copy(
                v_hbm.at[b, pl.ds(sb * SBLK, SBLK)],
                v_ref.at[b, pl.ds(sb * SBLK, SBLK)],
                kv_sems.at[NTILE + t]))
        if DO_KV:
            for cp in kv_copies:
                cp.start()

        barrier_sem = pltpu.get_barrier_semaphore()
        for nbr in (lax.rem(me + 1, N_DEV), lax.rem(me + N_DEV - 1, N_DEV)):
            pl.semaphore_signal(barrier_sem, inc=1, device_id=(nbr,),
                                device_id_type=pl.DeviceIdType.MESH)
        pl.semaphore_wait(barrier_sem, 2)

        stage_ref[pl.ds(me, 1)] = jnp.zeros((1, CHUNK, DM), jnp.float32)

        x2 = x_ref[...].reshape(ROWS, DM)
        qhs = [jnp.dot(x2, wq_ref[:, h * DH:(h + 1) * DH],
                       preferred_element_type=jnp.float32)
               for h in range(H_PER)]

        sel = (lax.broadcasted_iota(jnp.int32, (HQ, H_PER), 0)
               == me * H_PER
               + lax.broadcasted_iota(jnp.int32, (HQ, H_PER), 1)
               ).astype(jnp.float32)

        for t in range(NTILE):
            b, sb = divmod(t, NBLK)
            if DO_KV:
                kv_copies[2 * t].wait()
                kv_copies[2 * t + 1].wait()
            rows = pl.ds(sb * SBLK, SBLK)
            k4 = lax.dot_general(
                sel, k_ref[b, rows].reshape(SBLK * DH, HQ),
                (((0,), (1,)), ((), ())),
                preferred_element_type=jnp.float32,
            ).reshape(H_PER, SBLK, DH)
            v4 = lax.dot_general(
                sel, v_ref[b, rows].reshape(SBLK * DH, HQ),
                (((0,), (1,)), ((), ())),
                preferred_element_type=jnp.float32,
            ).reshape(H_PER, SBLK, DH)
            ctxs = []
            for h in range(H_PER):
                r0 = b * SQ + sb * SBLK
                qh = qhs[h][r0:r0 + SBLK]
                kh = k4[h]
                vh = v4[h]
                s = lax.dot_general(
                    qh, kh, (((1,), (1,)), ((), ())),
                    preferred_element_type=jnp.float32) * 0.125
                w = jnp.exp(s - jnp.max(s, axis=-1, keepdims=True))
                w = w / jnp.sum(w, axis=-1, keepdims=True)
                ctxs.append(jnp.dot(w, vh, preferred_element_type=jnp.float32))
            ctx_blk = jnp.concatenate(ctxs, axis=1)
            pb = jnp.dot(ctx_blk, wo_ref[...],
                         preferred_element_type=jnp.float32)
            c0 = b * (SQ // CHUNK) + sb * (SBLK // CHUNK)
            acc_ref[pl.ds(c0, SBLK // CHUNK)] = pb.reshape(
                SBLK // CHUNK, CHUNK, DM)

            if DO_RS:
                for c in range(c0, c0 + SBLK // CHUNK):

                    @pl.when(c != me)
                    def _send(c=c):
                        pltpu.make_async_remote_copy(
                            src_ref=acc_ref.at[c],
                            dst_ref=stage_ref.at[me],
                            send_sem=rs_send_sems.at[c],
                            recv_sem=rs_recv_sems.at[me],
                            device_id=(c,),
                            device_id_type=pl.DeviceIdType.MESH,
                        ).start()

        for off in range(1, N_DEV if DO_RS else 1):
            j = lax.rem(me + off, N_DEV)
            pltpu.make_async_remote_copy(
                src_ref=acc_ref.at[j],
                dst_ref=stage_ref.at[j],
                send_sem=rs_send_sems.at[j],
                recv_sem=rs_recv_sems.at[j],
                device_id=(j,),
                device_id_type=pl.DeviceIdType.MESH,
            ).wait_recv()
        red = (acc_ref[pl.ds(me, 1)]
               + jnp.sum(stage_ref[...], axis=0, keepdims=True))
        acc_ref[pl.ds(me, 1)] = red

        for off in range(1, N_DEV if DO_AG else 1):
            t = lax.rem(me + off, N_DEV)
            pltpu.make_async_remote_copy(
                src_ref=acc_ref.at[me],
                dst_ref=acc_ref.at[me],
                send_sem=ag_send_sems.at[t],
                recv_sem=ag_recv_sems.at[me],
                device_id=(t,),
                device_id_type=pl.DeviceIdType.MESH,
            ).start()

        for off in range(1, N_DEV if DO_AG else 1):
            j = lax.rem(me + off, N_DEV)
            pltpu.make_async_remote_copy(
                src_ref=acc_ref.at[j],
                dst_ref=acc_ref.at[j],
                send_sem=ag_send_sems.at[j],
                recv_sem=ag_recv_sems.at[j],
                device_id=(j,),
                device_id_type=pl.DeviceIdType.MESH,
            ).wait_recv()

        if DO_RS:
            for c in range(N_DEV):

                @pl.when(c != me)
                def _drain(c=c):
                    pltpu.make_async_remote_copy(
                        src_ref=acc_ref.at[c],
                        dst_ref=stage_ref.at[me],
                        send_sem=rs_send_sems.at[c],
                        recv_sem=rs_recv_sems.at[me],
                        device_id=(c,),
                        device_id_type=pl.DeviceIdType.MESH,
                    ).wait_send()
        if DO_AG:
            for off in range(1, N_DEV):
                t = lax.rem(me + off, N_DEV)
                pltpu.make_async_remote_copy(
                    src_ref=acc_ref.at[me],
                    dst_ref=acc_ref.at[me],
                    send_sem=ag_send_sems.at[t],
                    recv_sem=ag_recv_sems.at[me],
                    device_id=(t,),
                    device_id_type=pl.DeviceIdType.MESH,
                ).wait_send()

        out_ref[...] = acc_ref[...].reshape(B, SQ, DM)

    return pl.pallas_call(
        body,
        out_shape=jax.ShapeDtypeStruct((B, SQ, DM), jnp.float32),
        in_specs=[
            pl.BlockSpec(memory_space=pltpu.VMEM),
            pl.BlockSpec(memory_space=pltpu.VMEM),
            pl.BlockSpec(memory_space=pl.ANY),
            pl.BlockSpec(memory_space=pl.ANY),
            pl.BlockSpec(memory_space=pltpu.VMEM),
        ],
        out_specs=pl.BlockSpec(memory_space=pltpu.VMEM),
        scratch_shapes=[
            pltpu.VMEM((B, SQ, DH, HQ), jnp.float32),
            pltpu.VMEM((B, SQ, DH, HQ), jnp.float32),
            pltpu.VMEM((N_DEV, CHUNK, DM), jnp.float32),
            pltpu.VMEM((N_DEV, CHUNK, DM), jnp.float32),
            pltpu.SemaphoreType.DMA((2 * NTILE,)),
            pltpu.SemaphoreType.DMA((N_DEV,)),
            pltpu.SemaphoreType.DMA((N_DEV,)),
            pltpu.SemaphoreType.DMA((N_DEV,)),
            pltpu.SemaphoreType.DMA((N_DEV,)),
        ],
        compiler_params=pltpu.CompilerParams(
            collective_id=0,
            vmem_limit_bytes=50 * 1024 * 1024,
        ),
    )(x, Wq, K_t, V_t, Wo)
